# baseline (speedup 1.0000x reference)
import functools

import jax
import jax.numpy as jnp
from jax.experimental import pallas as pl

Q = 1024
D = 32
K_TOTAL = 100000
KT = 2048
K_PAD = ((K_TOTAL + KT - 1) // KT) * KT
NSTEP = K_PAD // KT


def _dist_body(q_ref, kt_ref, k_ref, out_ref):
    step = pl.program_id(0)
    q = q_ref[...]
    kt = kt_ref[...]
    k = k_ref[...]
    q_sq = jnp.sum(q * q, axis=1, keepdims=True)
    k_sq = jnp.sum(k * k, axis=1)
    dots = jax.lax.dot_general(
        q, kt, (((1,), (0,)), ((), ())),
        preferred_element_type=jnp.float32)
    dist = (q_sq - 2.0 * dots) + k_sq[None, :]
    col = step * KT + jax.lax.broadcasted_iota(jnp.int32, (Q, KT), 1)
    dist = jnp.where(col < K_TOTAL, dist, jnp.float32(3.0e38))
    out_ref[...] = dist


def kernel(queries, keys, k):
    keys_p = jnp.pad(keys, ((0, K_PAD - K_TOTAL), (0, 0)))
    keys_t = keys_p.T
    dist = pl.pallas_call(
        _dist_body,
        grid=(NSTEP,),
        in_specs=[
            pl.BlockSpec((Q, D), lambda i: (0, 0)),
            pl.BlockSpec((D, KT), lambda i: (0, i)),
            pl.BlockSpec((KT, D), lambda i: (i, 0)),
        ],
        out_specs=pl.BlockSpec((Q, KT), lambda i: (0, i)),
        out_shape=jax.ShapeDtypeStruct((Q, K_PAD), jnp.float32),
    )(queries, keys_t, keys_p)
    neg = -dist[:, :K_TOTAL]
    top_vals, top_idx = jax.lax.top_k(neg, queries.shape[1])
    gathered = jnp.take(keys, top_idx, axis=0)
    return gathered, -top_vals, top_idx


# speedup vs baseline: 1.8329x; 1.8329x over previous
import functools

import jax
import jax.numpy as jnp
from jax import lax
from jax.experimental import pallas as pl
from jax.experimental.pallas import tpu as pltpu
from jax.experimental.pallas import tpu_sc as plsc

Q = 1024
D = 32
K_TOTAL = 100000
KT = 1024
K_PAD = 100352
NSTEP = K_PAD // KT
CB = 16
NB = K_PAD // CB
CA = 256
NA = K_PAD // CA
TOPK = 32
NCAND = TOPK * CB
BIG = 3.0e38
IBIG = 2**31 - 1


def _s1_body(q_ref, kt_ref, k_ref, mb_ref, ma_ref):
    step = pl.program_id(0)
    q = q_ref[...]
    kt = kt_ref[...]
    k = k_ref[...]
    q_sq = jnp.sum(q * q, axis=1, keepdims=True)
    k_sq = jnp.sum(k * k, axis=1)
    dots = lax.dot_general(q, kt, (((1,), (0,)), ((), ())),
                           preferred_element_type=jnp.float32)
    dist = (q_sq - 2.0 * dots) + k_sq[None, :]
    col = step * KT + lax.broadcasted_iota(jnp.int32, (Q, KT), 1)
    dist = jnp.where(col < K_TOTAL, dist, BIG)
    bmin = jnp.min(dist.reshape(Q, KT // CB, CB), axis=2)
    mb_ref[0] = bmin
    ma_ref[0] = jnp.min(bmin.reshape(Q, KT // CA, CA // CB), axis=2)


def _s1(queries, keys_p, keys_t):
    return pl.pallas_call(
        _s1_body,
        grid=(NSTEP,),
        in_specs=[
            pl.BlockSpec((Q, D), lambda i: (0, 0)),
            pl.BlockSpec((D, KT), lambda i: (0, i)),
            pl.BlockSpec((KT, D), lambda i: (i, 0)),
        ],
        out_specs=[
            pl.BlockSpec((1, Q, KT // CB), lambda i: (i, 0, 0)),
            pl.BlockSpec((1, Q, KT // CA), lambda i: (i, 0, 0)),
        ],
        out_shape=[
            jax.ShapeDtypeStruct((NSTEP, Q, KT // CB), jnp.float32),
            jax.ShapeDtypeStruct((NSTEP, Q, KT // CA), jnp.float32),
        ],
    )(queries, keys_t, keys_p)


def _extract_body(v_ref, g_ref, outv_ref, outi_ref):
    V = v_ref[...]
    G = g_ref[...]
    for i in range(TOPK):
        m = jnp.min(V, axis=0)
        hit = V == m[None, :]
        gi = jnp.min(jnp.where(hit, G, IBIG), axis=0)
        outv_ref[i] = m
        outi_ref[i] = gi
        V = jnp.where(hit & (G == gi[None, :]), BIG, V)


def _extract(vals_t, ids_t):
    r = vals_t.shape[0]
    return pl.pallas_call(
        _extract_body,
        out_shape=[
            jax.ShapeDtypeStruct((TOPK, Q), jnp.float32),
            jax.ShapeDtypeStruct((TOPK, Q), jnp.int32),
        ],
    )(vals_t, ids_t)


def _sc_gather(table, idx, width, chunk):
    b = idx.shape[0]
    info = plsc.get_sparse_core_info()
    nw = info.num_cores * info.num_subcores
    n = b // nw
    mesh = plsc.VectorSubcoreMesh(core_axis_name="c", subcore_axis_name="s")

    @functools.partial(
        pl.kernel, mesh=mesh,
        compiler_params=pltpu.CompilerParams(use_tc_tiling_on_sc=False),
        out_type=jax.ShapeDtypeStruct((b, width), jnp.float32),
        scratch_types=[
            pltpu.VMEM((chunk,), jnp.int32),
            pltpu.VMEM((chunk, width), jnp.float32),
            pltpu.SemaphoreType.DMA,
        ],
    )
    def k(idx_hbm, table_hbm, out_hbm, idx_v, rows_v, sem):
        wid = lax.axis_index("s") * info.num_cores + lax.axis_index("c")
        base = wid * n
        for j in range(n // chunk):
            off = base + j * chunk
            pltpu.sync_copy(idx_hbm.at[pl.ds(off, chunk)], idx_v)
            pltpu.async_copy(table_hbm.at[idx_v], rows_v, sem).wait()
            pltpu.sync_copy(rows_v, out_hbm.at[pl.ds(off, chunk)])

    return k(idx, table)


QB = 8
CBLK = QB * NCAND


def _s6a_body(c_ref, q_ref, out_ref):
    cand = c_ref[...]
    qb = q_ref[...]
    k_sq = jnp.sum(cand * cand, axis=1)
    q_sq = jnp.sum(qb * qb, axis=1)
    dots = lax.dot_general(cand, qb, (((1,), (1,)), ((), ())),
                           preferred_element_type=jnp.float32)
    dist = (q_sq[None, :] - 2.0 * dots) + k_sq[:, None]
    qslot = lax.broadcasted_iota(jnp.int32, (CBLK, QB), 0) // NCAND
    colj = lax.broadcasted_iota(jnp.int32, (CBLK, QB), 1)
    own = jnp.sum(jnp.where(qslot == colj, dist, 0.0), axis=1)
    out_ref[...] = own.reshape(QB, NCAND)


def _s6a(cand, queries):
    return pl.pallas_call(
        _s6a_body,
        grid=(Q // QB,),
        in_specs=[
            pl.BlockSpec((CBLK, D), lambda i: (i, 0)),
            pl.BlockSpec((QB, D), lambda i: (i, 0)),
        ],
        out_specs=pl.BlockSpec((QB, NCAND), lambda i: (i, 0)),
        out_shape=jax.ShapeDtypeStruct((Q, NCAND), jnp.float32),
    )(cand, queries)


def kernel(queries, keys, k):
    keys_p = jnp.pad(keys, ((0, K_PAD - K_TOTAL), (0, 0)))
    keys_t = keys_p.T

    m_b3, m_a3 = _s1(queries, keys_p, keys_t)
    m_at = m_a3.transpose(0, 2, 1).reshape(NA, Q)

    a_ids = lax.broadcasted_iota(jnp.int32, (NA, Q), 0)
    _, sel_a_t = _extract(m_at, a_ids)
    sel_a = sel_a_t.T

    qcol = jnp.arange(Q, dtype=jnp.int32)[:, None]
    bm_idx = ((sel_a // 4) * 4096 + qcol * 4 + sel_a % 4).reshape(-1)
    bm = _sc_gather(m_b3.reshape(NSTEP * Q * 4, CB), bm_idx, CB, 1024)
    bm_t = bm.reshape(Q, NCAND).T

    gb = (sel_a[:, :, None] * CB +
          jnp.arange(CB, dtype=jnp.int32)[None, None, :]).reshape(Q, NCAND)
    _, sel_b_t = _extract(bm_t, gb.T)
    sel_b = sel_b_t.T

    ck = (sel_b[:, :, None] * CB +
          jnp.arange(CB, dtype=jnp.int32)[None, None, :]).reshape(Q, NCAND)
    cand = _sc_gather(keys_p, ck.reshape(-1), D, 2048)

    dist_c = _s6a(cand, queries)
    vals_t, idx_t = _extract(dist_c.T, ck.T)
    top_vals = vals_t.T
    top_idx = idx_t.T + (k * 0)

    gathered = _sc_gather(keys_p, top_idx.reshape(-1), D, 1024)
    return gathered.reshape(Q, TOPK, D), top_vals, top_idx
